# Initial kernel scaffold; baseline (speedup 1.0000x reference)
import functools

import jax
import jax.numpy as jnp
from jax import lax
from jax.experimental import pallas as pl
from jax.experimental.pallas import tpu as pltpu
from jax.experimental.pallas import tpu_sc as plsc

VOCAB = 65536
EMB = 32
ROWS = 16384
COLS = 200
N = ROWS * COLS
NW = 32
NPW = N // NW
C = 1024
NCHUNK = NPW // C
GSUB = C // 128

_mesh = plsc.VectorSubcoreMesh(core_axis_name="c", subcore_axis_name="s")


@functools.partial(
    pl.kernel,
    mesh=_mesh,
    out_type=(
        jax.ShapeDtypeStruct((N,), jnp.float32),
        jax.ShapeDtypeStruct((N * 8,), jnp.float32),
        jax.ShapeDtypeStruct((N, EMB), jnp.float32),
    ),
    scratch_types=[
        pltpu.VMEM((C,), jnp.int32),
        pltpu.VMEM((GSUB, 128), jnp.int32),
        pltpu.VMEM((C,), jnp.float32),
        pltpu.VMEM((C * 8,), jnp.float32),
        pltpu.VMEM((C, EMB), jnp.float32),
        pltpu.SemaphoreType.DMA,
    ],
)
def _qc_kernel(flags_hbm, table_hbm, mask_hbm, bits_hbm, emb_hbm,
               flags_v, ids_v, mask_v, bits_v, rows_v, sem):
    wid = lax.axis_index("s") * 2 + lax.axis_index("c")
    wbase = wid * NPW
    iota = lax.iota(jnp.int32, 16)
    pair_base = iota >> 3
    bit_pat = iota & 7

    def chunk(g, carry):
        base = wbase + g * C
        pltpu.sync_copy(flags_hbm.at[pl.ds(base, C)], flags_v)
        for j in range(GSUB):
            for c in range(8):
                off = j * 128 + c * 16
                f = flags_v[pl.ds(off, 16)]
                mask_v[pl.ds(off, 16)] = jnp.where(
                    (f & 7) == 0, 1.0, 0.0).astype(jnp.float32)
                ids_v[j, pl.ds(c * 16, 16)] = jnp.clip(f, 0, VOCAB - 1)
        copies = [
            pltpu.async_copy(table_hbm.at[ids_v.at[j]],
                             rows_v.at[pl.ds(j * 128, 128)], sem)
            for j in range(GSUB)
        ]

        def bits_body(j, _):
            fg = plsc.load_gather(flags_v, [pair_base + j * 2])
            b = ((fg >> bit_pat) & 1).astype(jnp.float32)
            bits_v[pl.ds(j * 16, 16)] = b
            return 0

        lax.fori_loop(0, C // 2, bits_body, 0)
        for cp in copies:
            cp.wait()
        pltpu.sync_copy(mask_v, mask_hbm.at[pl.ds(base, C)])
        pltpu.sync_copy(bits_v, bits_hbm.at[pl.ds(base * 8, C * 8)])
        pltpu.sync_copy(rows_v, emb_hbm.at[pl.ds(base, C)])
        return carry

    lax.fori_loop(0, NCHUNK, chunk, 0)


def kernel(qc_flags, emb_table):
    flags = qc_flags.reshape(N)
    mask, bits, emb = _qc_kernel(flags, emb_table)
    return (mask.reshape(ROWS, COLS),
            bits.reshape(ROWS, COLS, 8),
            emb.reshape(ROWS, COLS, EMB))


# speedup vs baseline: 4.0050x; 4.0050x over previous
import functools

import jax
import jax.numpy as jnp
from jax import lax
from jax.experimental import pallas as pl
from jax.experimental.pallas import tpu as pltpu
from jax.experimental.pallas import tpu_sc as plsc

VOCAB = 65536
EMB = 32
ROWS = 16384
COLS = 200
N = ROWS * COLS
NW = 32
NPW = N // NW
C = 1024
NCHUNK = NPW // C
GSUB = C // 128

_mesh = plsc.VectorSubcoreMesh(core_axis_name="c", subcore_axis_name="s")


@functools.partial(
    pl.kernel,
    mesh=_mesh,
    compiler_params=pltpu.CompilerParams(use_tc_tiling_on_sc=False),
    out_type=(
        jax.ShapeDtypeStruct((N,), jnp.float32),
        jax.ShapeDtypeStruct((N * 8,), jnp.float32),
        jax.ShapeDtypeStruct((N, EMB), jnp.float32),
    ),
    scratch_types=[
        pltpu.VMEM((C,), jnp.int32),
        pltpu.VMEM((GSUB, 128), jnp.int32),
        pltpu.VMEM((C,), jnp.float32),
        pltpu.VMEM((C * 8,), jnp.float32),
        pltpu.VMEM((C, EMB), jnp.float32),
        pltpu.SemaphoreType.DMA,
    ],
)
def _qc_kernel(flags_hbm, table_hbm, mask_hbm, bits_hbm, emb_hbm,
               flags_v, ids_v, mask_v, bits_v, rows_v, sem):
    wid = lax.axis_index("s") * 2 + lax.axis_index("c")
    wbase = wid * NPW
    iota = lax.iota(jnp.int32, 16)
    lane_lo = iota < 8
    bit_pat = iota & 7

    def chunk(g, carry):
        base = wbase + g * C
        pltpu.sync_copy(flags_hbm.at[pl.ds(base, C)], flags_v)
        for j in range(GSUB):
            for c in range(8):
                off = j * 128 + c * 16
                f = flags_v[pl.ds(off, 16)]
                mask_v[pl.ds(off, 16)] = jnp.where(
                    (f & 7) == 0, 1.0, 0.0).astype(jnp.float32)
                ids_v[j, pl.ds(c * 16, 16)] = jnp.clip(f, 0, VOCAB - 1)
        copies = [
            pltpu.async_copy(table_hbm.at[ids_v.at[j]],
                             rows_v.at[pl.ds(j * 128, 128)], sem)
            for j in range(GSUB)
        ]

        def bits_body(j, _):
            f = flags_v[pl.ds(j * 16, 16)]
            for k in range(8):
                f0 = jnp.full((16,), f[2 * k], jnp.int32)
                f1 = jnp.full((16,), f[2 * k + 1], jnp.int32)
                fv = jnp.where(lane_lo, f0, f1)
                b = ((fv >> bit_pat) & 1).astype(jnp.float32)
                bits_v[pl.ds(j * 128 + k * 16, 16)] = b
            return 0

        lax.fori_loop(0, C // 16, bits_body, 0)
        for cp in copies:
            cp.wait()
        pltpu.sync_copy(mask_v, mask_hbm.at[pl.ds(base, C)])
        pltpu.sync_copy(bits_v, bits_hbm.at[pl.ds(base * 8, C * 8)])
        pltpu.sync_copy(rows_v, emb_hbm.at[pl.ds(base, C)])
        return carry

    lax.fori_loop(0, NCHUNK, chunk, 0)


def kernel(qc_flags, emb_table):
    flags = qc_flags.reshape(N)
    mask, bits, emb = _qc_kernel(flags, emb_table)
    return (mask.reshape(ROWS, COLS),
            bits.reshape(ROWS, COLS, 8),
            emb.reshape(ROWS, COLS, EMB))


# speedup vs baseline: 4.1691x; 1.0410x over previous
import functools

import jax
import jax.numpy as jnp
from jax import lax
from jax.experimental import pallas as pl
from jax.experimental.pallas import tpu as pltpu
from jax.experimental.pallas import tpu_sc as plsc

VOCAB = 65536
EMB = 32
ROWS = 16384
COLS = 200
N = ROWS * COLS
NW = 32
NPW = N // NW
C = 1024
NCHUNK = NPW // C
GSUB = C // 128

_mesh = plsc.VectorSubcoreMesh(core_axis_name="c", subcore_axis_name="s")


@functools.partial(
    pl.kernel,
    mesh=_mesh,
    compiler_params=pltpu.CompilerParams(use_tc_tiling_on_sc=False),
    out_type=(
        jax.ShapeDtypeStruct((N,), jnp.float32),
        jax.ShapeDtypeStruct((N * 8,), jnp.float32),
        jax.ShapeDtypeStruct((N, EMB), jnp.float32),
    ),
    scratch_types=[
        pltpu.VMEM((C,), jnp.int32),
        pltpu.VMEM((C,), jnp.int32),
        pltpu.VMEM((GSUB, 128), jnp.int32),
        pltpu.VMEM((GSUB, 128), jnp.int32),
        pltpu.VMEM((C,), jnp.float32),
        pltpu.VMEM((C,), jnp.float32),
        pltpu.VMEM((C * 8,), jnp.float32),
        pltpu.VMEM((C * 8,), jnp.float32),
        pltpu.VMEM((C, EMB), jnp.float32),
        pltpu.VMEM((C, EMB), jnp.float32),
        pltpu.SemaphoreType.DMA,
        pltpu.SemaphoreType.DMA,
        pltpu.SemaphoreType.DMA,
        pltpu.SemaphoreType.DMA,
        pltpu.SemaphoreType.DMA,
        pltpu.SemaphoreType.DMA,
    ],
)
def _qc_kernel(flags_hbm, table_hbm, mask_hbm, bits_hbm, emb_hbm,
               flags_v0, flags_v1, ids_v0, ids_v1, mask_v0, mask_v1,
               bits_v0, bits_v1, rows_v0, rows_v1,
               sem_i0, sem_i1, sem_g0, sem_g1, sem_o0, sem_o1):
    flags_b = (flags_v0, flags_v1)
    ids_b = (ids_v0, ids_v1)
    mask_b = (mask_v0, mask_v1)
    bits_b = (bits_v0, bits_v1)
    rows_b = (rows_v0, rows_v1)
    sem_i = (sem_i0, sem_i1)
    sem_g = (sem_g0, sem_g1)
    sem_o = (sem_o0, sem_o1)

    wid = lax.axis_index("s") * 2 + lax.axis_index("c")
    wbase = wid * NPW
    iota = lax.iota(jnp.int32, 16)
    lane_lo = iota < 8
    bit_pat = iota & 7

    def in_copy(g, p):
        return pltpu.make_async_copy(
            flags_hbm.at[pl.ds(wbase + g * C, C)], flags_b[p], sem_i[p])

    def gather_copy(p, j):
        return pltpu.make_async_copy(
            table_hbm.at[ids_b[p].at[j]],
            rows_b[p].at[pl.ds(j * 128, 128)], sem_g[p])

    def out_copies(g, p):
        base = wbase + g * C
        return (
            pltpu.make_async_copy(
                mask_b[p], mask_hbm.at[pl.ds(base, C)], sem_o[p]),
            pltpu.make_async_copy(
                bits_b[p], bits_hbm.at[pl.ds(base * 8, C * 8)], sem_o[p]),
            pltpu.make_async_copy(
                rows_b[p], emb_hbm.at[pl.ds(base, C)], sem_o[p]),
        )

    in_copy(0, 0).start()

    def outer(g2, carry):
        for p in range(2):
            g = g2 * 2 + p
            in_copy(g, p).wait()

            @pl.when(g2 >= 1)
            def _drain():
                for cp in out_copies(g - 2, p):
                    cp.wait()

            for j in range(GSUB):
                for c in range(8):
                    off = j * 128 + c * 16
                    f = flags_b[p][pl.ds(off, 16)]
                    mask_b[p][pl.ds(off, 16)] = jnp.where(
                        (f & 7) == 0, 1.0, 0.0).astype(jnp.float32)
                    ids_b[p][j, pl.ds(c * 16, 16)] = jnp.clip(f, 0, VOCAB - 1)

            for j in range(GSUB):
                gather_copy(p, j).start()

            @pl.when(g + 1 < NCHUNK)
            def _prefetch():
                in_copy(g + 1, 1 - p).start()

            def bits_body(j, _, p=p):
                f = flags_b[p][pl.ds(j * 16, 16)]
                for k in range(8):
                    f0 = jnp.full((16,), f[2 * k], jnp.int32)
                    f1 = jnp.full((16,), f[2 * k + 1], jnp.int32)
                    fv = jnp.where(lane_lo, f0, f1)
                    b = ((fv >> bit_pat) & 1).astype(jnp.float32)
                    bits_b[p][pl.ds(j * 128 + k * 16, 16)] = b
                return 0

            lax.fori_loop(0, C // 16, bits_body, 0)

            for j in range(GSUB):
                gather_copy(p, j).wait()
            for cp in out_copies(g, p):
                cp.start()
        return carry

    lax.fori_loop(0, NCHUNK // 2, outer, 0)
    for p in range(2):
        for cp in out_copies(NCHUNK - 2 + p, p):
            cp.wait()


def kernel(qc_flags, emb_table):
    flags = qc_flags.reshape(N)
    mask, bits, emb = _qc_kernel(flags, emb_table)
    return (mask.reshape(ROWS, COLS),
            bits.reshape(ROWS, COLS, 8),
            emb.reshape(ROWS, COLS, EMB))


# speedup vs baseline: 6.4548x; 1.5482x over previous
import functools

import jax
import jax.numpy as jnp
from jax import lax
from jax.experimental import pallas as pl
from jax.experimental.pallas import tpu as pltpu
from jax.experimental.pallas import tpu_sc as plsc

VOCAB = 65536
EMB = 32
ROWS = 16384
COLS = 200
N = ROWS * COLS
NW = 32
NPW = N // NW
C = 1024
NCHUNK = NPW // C
GSUB = C // 128

_mesh = plsc.VectorSubcoreMesh(core_axis_name="c", subcore_axis_name="s")


@functools.partial(
    pl.kernel,
    mesh=_mesh,
    compiler_params=pltpu.CompilerParams(use_tc_tiling_on_sc=False),
    out_type=jax.ShapeDtypeStruct((N, EMB), jnp.float32),
    scratch_types=[
        pltpu.VMEM((C,), jnp.int32),
        pltpu.VMEM((C,), jnp.int32),
        pltpu.VMEM((GSUB, 128), jnp.int32),
        pltpu.VMEM((GSUB, 128), jnp.int32),
        pltpu.VMEM((C, EMB), jnp.float32),
        pltpu.VMEM((C, EMB), jnp.float32),
        pltpu.SemaphoreType.DMA,
        pltpu.SemaphoreType.DMA,
        pltpu.SemaphoreType.DMA,
        pltpu.SemaphoreType.DMA,
        pltpu.SemaphoreType.DMA,
        pltpu.SemaphoreType.DMA,
    ],
)
def _gather_kernel(flags_hbm, table_hbm, emb_hbm,
                   flags_v0, flags_v1, ids_v0, ids_v1, rows_v0, rows_v1,
                   sem_i0, sem_i1, sem_g0, sem_g1, sem_o0, sem_o1):
    flags_b = (flags_v0, flags_v1)
    ids_b = (ids_v0, ids_v1)
    rows_b = (rows_v0, rows_v1)
    sem_i = (sem_i0, sem_i1)
    sem_g = (sem_g0, sem_g1)
    sem_o = (sem_o0, sem_o1)

    wid = lax.axis_index("s") * 2 + lax.axis_index("c")
    wbase = wid * NPW

    def in_copy(g, p):
        return pltpu.make_async_copy(
            flags_hbm.at[pl.ds(wbase + g * C, C)], flags_b[p], sem_i[p])

    def gather_copy(p, j):
        return pltpu.make_async_copy(
            table_hbm.at[ids_b[p].at[j]],
            rows_b[p].at[pl.ds(j * 128, 128)], sem_g[p])

    def out_copy(g, p):
        return pltpu.make_async_copy(
            rows_b[p], emb_hbm.at[pl.ds(wbase + g * C, C)], sem_o[p])

    in_copy(0, 0).start()

    def outer(g2, carry):
        for p in range(2):
            g = g2 * 2 + p
            in_copy(g, p).wait()

            @pl.when(g2 >= 1)
            def _drain():
                out_copy(g - 2, p).wait()

            for j in range(GSUB):
                for c in range(8):
                    f = flags_b[p][pl.ds(j * 128 + c * 16, 16)]
                    ids_b[p][j, pl.ds(c * 16, 16)] = jnp.clip(f, 0, VOCAB - 1)

            for j in range(GSUB):
                gather_copy(p, j).start()

            @pl.when(g + 1 < NCHUNK)
            def _prefetch():
                in_copy(g + 1, 1 - p).start()

            for j in range(GSUB):
                gather_copy(p, j).wait()
            out_copy(g, p).start()
        return carry

    lax.fori_loop(0, NCHUNK // 2, outer, 0)
    for p in range(2):
        out_copy(NCHUNK - 2 + p, p).wait()


RPW = ROWS // NW
RB = 512
CSTRIPES = COLS // 8


@functools.partial(
    pl.kernel,
    mesh=_mesh,
    compiler_params=pltpu.CompilerParams(use_tc_tiling_on_sc=True),
    out_type=(
        jax.ShapeDtypeStruct((COLS, ROWS), jnp.float32),
        jax.ShapeDtypeStruct((COLS, 8, ROWS), jnp.float32),
    ),
    scratch_types=[
        pltpu.VMEM((8, RB), jnp.int32),
        pltpu.VMEM((8, RB), jnp.float32),
        pltpu.VMEM((8, 8, RB), jnp.float32),
    ],
)
def _maskbits_kernel(flagst_hbm, maskt_hbm, bitst_hbm, flags_v, mask_v, bits_v):
    wid = lax.axis_index("s") * 2 + lax.axis_index("c")
    rbase = wid * RPW

    def stripe(i, carry):
        c0 = i * 8
        pltpu.sync_copy(
            flagst_hbm.at[pl.ds(c0, 8), pl.ds(rbase, RB)], flags_v)
        for c in range(8):
            for v in range(RB // 16):
                f = flags_v[c, pl.ds(v * 16, 16)]
                mask_v[c, pl.ds(v * 16, 16)] = jnp.where(
                    (f & 7) == 0, 1.0, 0.0).astype(jnp.float32)
                for b in range(8):
                    bits_v[c, b, pl.ds(v * 16, 16)] = (
                        (f >> b) & 1).astype(jnp.float32)
        pltpu.sync_copy(
            mask_v, maskt_hbm.at[pl.ds(c0, 8), pl.ds(rbase, RB)])
        pltpu.sync_copy(
            bits_v, bitst_hbm.at[pl.ds(c0, 8), :, pl.ds(rbase, RB)])
        return carry

    lax.fori_loop(0, CSTRIPES, stripe, 0)


def kernel(qc_flags, emb_table):
    flags = qc_flags.reshape(N)
    emb = _gather_kernel(flags, emb_table)
    mask_t, bits_t = _maskbits_kernel(qc_flags.T)
    return (mask_t.T,
            jnp.transpose(bits_t, (2, 0, 1)),
            emb.reshape(ROWS, COLS, EMB))


# speedup vs baseline: 7.2533x; 1.1237x over previous
import functools

import jax
import jax.numpy as jnp
from jax import lax
from jax.experimental import pallas as pl
from jax.experimental.pallas import tpu as pltpu
from jax.experimental.pallas import tpu_sc as plsc

VOCAB = 65536
EMB = 32
ROWS = 16384
COLS = 200
N = ROWS * COLS
NW = 32
NPW = N // NW
C = 1024
NCHUNK = NPW // C
GSUB = C // 128

_mesh = plsc.VectorSubcoreMesh(core_axis_name="c", subcore_axis_name="s")


@functools.partial(
    pl.kernel,
    mesh=_mesh,
    compiler_params=pltpu.CompilerParams(use_tc_tiling_on_sc=False),
    out_type=jax.ShapeDtypeStruct((N, EMB), jnp.float32),
    scratch_types=[
        pltpu.VMEM((C,), jnp.int32),
        pltpu.VMEM((C,), jnp.int32),
        pltpu.VMEM((GSUB, 128), jnp.int32),
        pltpu.VMEM((GSUB, 128), jnp.int32),
        pltpu.VMEM((C, EMB), jnp.float32),
        pltpu.VMEM((C, EMB), jnp.float32),
        pltpu.SemaphoreType.DMA,
        pltpu.SemaphoreType.DMA,
        pltpu.SemaphoreType.DMA,
        pltpu.SemaphoreType.DMA,
        pltpu.SemaphoreType.DMA,
        pltpu.SemaphoreType.DMA,
    ],
)
def _gather_kernel(flags_hbm, table_hbm, emb_hbm,
                   flags_v0, flags_v1, ids_v0, ids_v1, rows_v0, rows_v1,
                   sem_i0, sem_i1, sem_g0, sem_g1, sem_o0, sem_o1):
    flags_b = (flags_v0, flags_v1)
    ids_b = (ids_v0, ids_v1)
    rows_b = (rows_v0, rows_v1)
    sem_i = (sem_i0, sem_i1)
    sem_g = (sem_g0, sem_g1)
    sem_o = (sem_o0, sem_o1)

    wid = lax.axis_index("s") * 2 + lax.axis_index("c")
    wbase = wid * NPW

    def in_copy(g, p):
        return pltpu.make_async_copy(
            flags_hbm.at[pl.ds(wbase + g * C, C)], flags_b[p], sem_i[p])

    def gather_copy(p, j):
        return pltpu.make_async_copy(
            table_hbm.at[ids_b[p].at[j]],
            rows_b[p].at[pl.ds(j * 128, 128)], sem_g[p])

    def out_copy(g, p):
        return pltpu.make_async_copy(
            rows_b[p], emb_hbm.at[pl.ds(wbase + g * C, C)], sem_o[p])

    in_copy(0, 0).start()

    def outer(g2, carry):
        for p in range(2):
            g = g2 * 2 + p
            in_copy(g, p).wait()

            @pl.when(g2 >= 1)
            def _drain():
                out_copy(g - 2, p).wait()

            for j in range(GSUB):
                for c in range(8):
                    f = flags_b[p][pl.ds(j * 128 + c * 16, 16)]
                    ids_b[p][j, pl.ds(c * 16, 16)] = jnp.clip(f, 0, VOCAB - 1)

            for j in range(GSUB):
                gather_copy(p, j).start()

            @pl.when(g + 1 < NCHUNK)
            def _prefetch():
                in_copy(g + 1, 1 - p).start()

            for j in range(GSUB):
                gather_copy(p, j).wait()
            out_copy(g, p).start()
        return carry

    lax.fori_loop(0, NCHUNK // 2, outer, 0)
    for p in range(2):
        out_copy(NCHUNK - 2 + p, p).wait()


RBLK = 2048
IBLK = RBLK // 4


def _emb_t_body(in_ref, out_ref):
    y = in_ref[...]
    z = y.reshape(RBLK, EMB)
    out_ref[...] = z.T.reshape(1, EMB, RBLK)


_emb_transpose = pl.pallas_call(
    _emb_t_body,
    grid=(COLS, ROWS // RBLK),
    in_specs=[pl.BlockSpec((1, IBLK, 128), lambda i, j: (i, j, 0))],
    out_specs=pl.BlockSpec((1, EMB, RBLK), lambda i, j: (i, 0, j)),
    out_shape=jax.ShapeDtypeStruct((COLS, EMB, ROWS), jnp.float32),
)


RPW = ROWS // NW
RB = 512
CSTRIPES = COLS // 8


@functools.partial(
    pl.kernel,
    mesh=_mesh,
    compiler_params=pltpu.CompilerParams(use_tc_tiling_on_sc=True),
    out_type=(
        jax.ShapeDtypeStruct((COLS, ROWS), jnp.float32),
        jax.ShapeDtypeStruct((COLS, 8, ROWS), jnp.float32),
    ),
    scratch_types=[
        pltpu.VMEM((8, RB), jnp.int32),
        pltpu.VMEM((8, RB), jnp.float32),
        pltpu.VMEM((8, 8, RB), jnp.float32),
    ],
)
def _maskbits_kernel(flagst_hbm, maskt_hbm, bitst_hbm, flags_v, mask_v, bits_v):
    wid = lax.axis_index("s") * 2 + lax.axis_index("c")
    rbase = wid * RPW

    def stripe(i, carry):
        c0 = i * 8
        pltpu.sync_copy(
            flagst_hbm.at[pl.ds(c0, 8), pl.ds(rbase, RB)], flags_v)
        for c in range(8):
            for v in range(RB // 16):
                f = flags_v[c, pl.ds(v * 16, 16)]
                mask_v[c, pl.ds(v * 16, 16)] = jnp.where(
                    (f & 7) == 0, 1.0, 0.0).astype(jnp.float32)
                for b in range(8):
                    bits_v[c, b, pl.ds(v * 16, 16)] = (
                        (f >> b) & 1).astype(jnp.float32)
        pltpu.sync_copy(
            mask_v, maskt_hbm.at[pl.ds(c0, 8), pl.ds(rbase, RB)])
        pltpu.sync_copy(
            bits_v, bitst_hbm.at[pl.ds(c0, 8), :, pl.ds(rbase, RB)])
        return carry

    lax.fori_loop(0, CSTRIPES, stripe, 0)


def kernel(qc_flags, emb_table):
    flags_t = qc_flags.T
    flags_c = flags_t.reshape(N)
    emb_g = _gather_kernel(flags_c, emb_table)
    mask_t, bits_t = _maskbits_kernel(flags_t)
    return (mask_t.T,
            jnp.transpose(bits_t, (2, 0, 1)),
            jnp.transpose(emb_g.reshape(COLS, ROWS, EMB), (1, 0, 2)))
